# baseline (speedup 1.0000x reference)
import functools

import jax
import jax.numpy as jnp
from jax import lax
from jax.experimental import pallas as pl
from jax.experimental.pallas import tpu as pltpu
from jax.experimental.pallas import tpu_sc as plsc

B = 16384
D = 32
N_CORES = 2
N_SUBCORES = 16
NW = N_CORES * N_SUBCORES
LANES = 16
NBUF = 8

TC_N = 4096
SC_N = B - TC_N
BPW = SC_N // NW

_mesh = plsc.VectorSubcoreMesh(core_axis_name="c", subcore_axis_name="s")


@functools.partial(
    pl.kernel,
    mesh=_mesh,
    out_type=jax.ShapeDtypeStruct((SC_N,), jnp.float32),
    compiler_params=pltpu.CompilerParams(
        needs_layout_passes=False, use_tc_tiling_on_sc=True),
    scratch_types=[
        pltpu.VMEM((BPW + LANES,), jnp.int32),
        pltpu.VMEM((BPW + LANES,), jnp.int32),
        pltpu.VMEM((NBUF, D, 128), jnp.float32),
        pltpu.VMEM((NBUF, D, 128), jnp.float32),
        pltpu.VMEM((BPW,), jnp.float32),
        pltpu.SemaphoreType.DMA,
        pltpu.SemaphoreType.DMA,
    ],
)
def _gcmc_sc_kernel(uid_hbm, iid_hbm, utT_hbm, itT_hbm, out_hbm,
                    uidx, iidx, ublk, iblk, out_v, usem, isem):
    wid = lax.axis_index("s") * N_CORES + lax.axis_index("c")
    base = wid * BPW

    pltpu.sync_copy(uid_hbm.at[pl.ds(base, BPW)], uidx.at[pl.ds(0, BPW)])
    pltpu.sync_copy(iid_hbm.at[pl.ds(base, BPW)], iidx.at[pl.ds(0, BPW)])

    lane = jnp.arange(LANES, dtype=jnp.int32)

    def issue(uvi, ivi, slot):
        tc_u = pl.multiple_of((uvi >> 7) * 128, 128)
        tc_i = pl.multiple_of((ivi >> 7) * 128, 128)
        pltpu.make_async_copy(
            utT_hbm.at[:, pl.ds(tc_u, 128)], ublk.at[slot], usem).start()
        pltpu.make_async_copy(
            itT_hbm.at[:, pl.ds(tc_i, 128)], iblk.at[slot], isem).start()

    def wait(slot):
        pltpu.make_async_copy(
            utT_hbm.at[:, pl.ds(0, 128)], ublk.at[slot], usem).wait()
        pltpu.make_async_copy(
            itT_hbm.at[:, pl.ds(0, 128)], iblk.at[slot], isem).wait()

    uvec0 = uidx[pl.ds(0, LANES)]
    ivec0 = iidx[pl.ds(0, LANES)]
    for n in range(NBUF):
        issue(uvec0[n], ivec0[n], n)

    def body(g, carry):
        uvec = uidx[pl.ds(g * LANES, LANES)]
        ivec = iidx[pl.ds(g * LANES, LANES)]
        uvec_n = uidx[pl.ds((g + 1) * LANES, LANES)]
        ivec_n = iidx[pl.ds((g + 1) * LANES, LANES)]
        acc = jnp.zeros((LANES,), jnp.float32)
        for i in range(LANES):
            slot = i % NBUF
            lu = jnp.full((LANES,), uvec[i] & 127, jnp.int32)
            li = jnp.full((LANES,), ivec[i] & 127, jnp.int32)

            wait(slot)
            u0 = plsc.load_gather(ublk.at[slot], [lane, lu])
            u1 = plsc.load_gather(ublk.at[slot], [lane + LANES, lu])
            v0 = plsc.load_gather(iblk.at[slot], [lane, li])
            v1 = plsc.load_gather(iblk.at[slot], [lane + LANES, li])
            s = jnp.sum(u0 * v0 + u1 * v1)
            acc = jnp.where(lane == i, s, acc)

            if i + NBUF < LANES:
                issue(uvec[i + NBUF], ivec[i + NBUF], slot)
            else:
                @pl.when(g < BPW // LANES - 1)
                def _():
                    issue(uvec_n[i + NBUF - LANES],
                          ivec_n[i + NBUF - LANES], slot)

        out_v[pl.ds(g * LANES, LANES)] = acc
        return carry

    lax.fori_loop(0, BPW // LANES, body, 0)

    pltpu.sync_copy(out_v, out_hbm.at[pl.ds(base, BPW)])


def _gcmc_tc_body(uid_s, iid_s, utT_hbm, itT_hbm, out_ref,
                  ublk, iblk, usem, isem):
    r_iota = lax.broadcasted_iota(jnp.int32, (D, 128), 0)
    c_iota = lax.broadcasted_iota(jnp.int32, (D, 128), 1)
    o_r = lax.broadcasted_iota(jnp.int32, (TC_N // 128, 128), 0)
    o_c = lax.broadcasted_iota(jnp.int32, (TC_N // 128, 128), 1)

    def issue(n, slot):
        tc_u = pl.multiple_of((uid_s[n] >> 7) * 128, 128)
        tc_i = pl.multiple_of((iid_s[n] >> 7) * 128, 128)
        pltpu.make_async_copy(
            utT_hbm.at[:, pl.ds(tc_u, 128)], ublk.at[slot], usem).start()
        pltpu.make_async_copy(
            itT_hbm.at[:, pl.ds(tc_i, 128)], iblk.at[slot], isem).start()

    def wait(slot):
        pltpu.make_async_copy(
            utT_hbm.at[:, pl.ds(0, 128)], ublk.at[slot], usem).wait()
        pltpu.make_async_copy(
            itT_hbm.at[:, pl.ds(0, 128)], iblk.at[slot], isem).wait()

    for n in range(NBUF):
        issue(n, n)

    def body(n, acc):
        slot = n % NBUF
        wait(slot)
        u2d = ublk[slot]
        i2d = iblk[slot]
        lu = uid_s[n] & 127
        li = iid_s[n] & 127
        ucol = jnp.sum(jnp.where(c_iota == lu, u2d, 0.0), axis=1,
                       keepdims=True)
        icol = jnp.sum(jnp.where(c_iota == li, i2d, 0.0), axis=1,
                       keepdims=True)
        s = jnp.sum(ucol * icol)

        @pl.when(n < TC_N - NBUF)
        def _():
            issue(n + NBUF, slot)

        return jnp.where((o_r == n // 128) & (o_c == n % 128), s, acc)

    acc = lax.fori_loop(
        0, TC_N, body, jnp.zeros((TC_N // 128, 128), jnp.float32))
    out_ref[...] = acc


_gcmc_tc_kernel = pl.pallas_call(
    _gcmc_tc_body,
    out_shape=jax.ShapeDtypeStruct((TC_N // 128, 128), jnp.float32),
    in_specs=[
        pl.BlockSpec(memory_space=pltpu.SMEM),
        pl.BlockSpec(memory_space=pltpu.SMEM),
        pl.BlockSpec(memory_space=pl.ANY),
        pl.BlockSpec(memory_space=pl.ANY),
    ],
    out_specs=pl.BlockSpec(memory_space=pltpu.VMEM),
    scratch_shapes=[
        pltpu.VMEM((NBUF, D, 128), jnp.float32),
        pltpu.VMEM((NBUF, D, 128), jnp.float32),
        pltpu.SemaphoreType.DMA,
        pltpu.SemaphoreType.DMA,
    ],
)


def kernel(x, user_embedding, item_embedding):
    uid = x[:, 0]
    iid = x[:, 1]
    utT = user_embedding.T
    itT = item_embedding.T
    out_sc = _gcmc_sc_kernel(uid[:SC_N], iid[:SC_N], utT, itT)
    out_tc = _gcmc_tc_kernel(uid[SC_N:], iid[SC_N:], utT, itT)
    return jnp.concatenate([out_sc, out_tc.reshape(TC_N)])


# speedup vs baseline: 4.3975x; 4.3975x over previous
import functools

import jax
import jax.numpy as jnp
from jax import lax
from jax.experimental import pallas as pl
from jax.experimental.pallas import tpu as pltpu
from jax.experimental.pallas import tpu_sc as plsc

B = 16384
D = 32
N_CORES = 2
N_SUBCORES = 16
NW = N_CORES * N_SUBCORES
BPW = B // NW
LANES = 16
NBUF = 8

_mesh = plsc.VectorSubcoreMesh(core_axis_name="c", subcore_axis_name="s")


@functools.partial(
    pl.kernel,
    mesh=_mesh,
    out_type=jax.ShapeDtypeStruct((B,), jnp.float32),
    compiler_params=pltpu.CompilerParams(
        needs_layout_passes=False, use_tc_tiling_on_sc=True),
    scratch_types=[
        pltpu.VMEM((BPW + LANES,), jnp.int32),
        pltpu.VMEM((BPW + LANES,), jnp.int32),
        pltpu.VMEM((NBUF, D, 128), jnp.float32),
        pltpu.VMEM((NBUF, D, 128), jnp.float32),
        pltpu.VMEM((BPW,), jnp.float32),
        pltpu.SemaphoreType.DMA,
        pltpu.SemaphoreType.DMA,
    ],
)
def _gcmc_sc_kernel(uid_hbm, iid_hbm, utT_hbm, itT_hbm, out_hbm,
                    uidx, iidx, ublk, iblk, out_v, usem, isem):
    wid = lax.axis_index("s") * N_CORES + lax.axis_index("c")
    base = wid * BPW

    pltpu.sync_copy(uid_hbm.at[pl.ds(base, BPW)], uidx.at[pl.ds(0, BPW)])
    pltpu.sync_copy(iid_hbm.at[pl.ds(base, BPW)], iidx.at[pl.ds(0, BPW)])

    lane = jnp.arange(LANES, dtype=jnp.int32)

    def issue(uvi, ivi, slot):
        tc_u = pl.multiple_of((uvi >> 7) * 128, 128)
        tc_i = pl.multiple_of((ivi >> 7) * 128, 128)
        pltpu.make_async_copy(
            utT_hbm.at[:, pl.ds(tc_u, 128)], ublk.at[slot], usem).start()
        pltpu.make_async_copy(
            itT_hbm.at[:, pl.ds(tc_i, 128)], iblk.at[slot], isem).start()

    def wait(slot):
        pltpu.make_async_copy(
            utT_hbm.at[:, pl.ds(0, 128)], ublk.at[slot], usem).wait()
        pltpu.make_async_copy(
            itT_hbm.at[:, pl.ds(0, 128)], iblk.at[slot], isem).wait()

    uvec0 = uidx[pl.ds(0, LANES)]
    ivec0 = iidx[pl.ds(0, LANES)]
    for n in range(NBUF):
        issue(uvec0[n], ivec0[n], n)

    def body(g, carry):
        uvec = uidx[pl.ds(g * LANES, LANES)]
        ivec = iidx[pl.ds(g * LANES, LANES)]
        uvec_n = uidx[pl.ds((g + 1) * LANES, LANES)]
        ivec_n = iidx[pl.ds((g + 1) * LANES, LANES)]
        acc = jnp.zeros((LANES,), jnp.float32)
        for i in range(LANES):
            slot = i % NBUF
            lu = jnp.full((LANES,), uvec[i] & 127, jnp.int32)
            li = jnp.full((LANES,), ivec[i] & 127, jnp.int32)

            wait(slot)
            u0 = plsc.load_gather(ublk.at[slot], [lane, lu])
            u1 = plsc.load_gather(ublk.at[slot], [lane + LANES, lu])
            v0 = plsc.load_gather(iblk.at[slot], [lane, li])
            v1 = plsc.load_gather(iblk.at[slot], [lane + LANES, li])
            s = jnp.sum(u0 * v0 + u1 * v1)
            acc = jnp.where(lane == i, s, acc)

            if i + NBUF < LANES:
                issue(uvec[i + NBUF], ivec[i + NBUF], slot)
            else:
                @pl.when(g < BPW // LANES - 1)
                def _():
                    issue(uvec_n[i + NBUF - LANES],
                          ivec_n[i + NBUF - LANES], slot)

        out_v[pl.ds(g * LANES, LANES)] = acc
        return carry

    lax.fori_loop(0, BPW // LANES, body, 0)

    pltpu.sync_copy(out_v, out_hbm.at[pl.ds(base, BPW)])


def kernel(x, user_embedding, item_embedding):
    uid = x[:, 0]
    iid = x[:, 1]
    return _gcmc_sc_kernel(uid, iid, user_embedding.T, item_embedding.T)
